# baseline (speedup 1.0000x reference)
import functools

import jax
import jax.numpy as jnp
from jax import lax
from jax.experimental import pallas as pl
from jax.experimental.pallas import tpu as pltpu
from jax.experimental.pallas import tpu_sc as plsc

_N = 1 << 24
_NA = 4
_NW = 32
_WCHUNK = _N // _NW
_TILE = 32768
_NT = _WCHUNK // _TILE
_UNROLL = 16

_mesh = plsc.VectorSubcoreMesh(core_axis_name="c", subcore_axis_name="s")


@functools.partial(
    pl.kernel,
    mesh=_mesh,
    out_type=jax.ShapeDtypeStruct((_NA, 128), jnp.float32),
    scratch_types=[
        pltpu.VMEM((2, _TILE), jnp.float32),
        pltpu.VMEM((16,), jnp.float32),
        pltpu.SemaphoreType.DMA,
        pltpu.SemaphoreType.DMA,
    ],
)
def _sc_segsum(probs_hbm, out_hbm, buf, accv, sem0, sem1):
    cid = lax.axis_index("c")
    sid = lax.axis_index("s")
    wid = cid * 16 + sid
    base = wid * _WCHUNK
    sems = (sem0, sem1)

    copies = [None, None]
    copies[0] = pltpu.async_copy(
        probs_hbm.at[pl.ds(base, _TILE)], buf.at[0], sems[0])

    acc = tuple(jnp.zeros((16,), jnp.float32) for _ in range(_UNROLL))

    def _tile_sum(d, acc):
        tile = buf.at[d]

        def body(it, a):
            o = it * (_UNROLL * 16)
            return tuple(
                a[j] + tile[pl.ds(o + j * 16, 16)] for j in range(_UNROLL))

        return lax.fori_loop(0, _TILE // (16 * _UNROLL), body, acc)

    for t in range(_NT):
        d = t % 2
        if t + 1 < _NT:
            nd = (t + 1) % 2
            copies[nd] = pltpu.async_copy(
                probs_hbm.at[pl.ds(base + (t + 1) * _TILE, _TILE)],
                buf.at[nd], sems[nd])
        copies[d].wait()
        acc = _tile_sum(d, acc)

    vec = acc[0]
    for j in range(1, _UNROLL):
        vec = vec + acc[j]
    accv[...] = vec

    b = wid // 8
    lane = (wid % 8) * 16
    pltpu.sync_copy(accv, out_hbm.at[b, pl.ds(lane, 16)])


def _finish_body(p_ref, o_ref):
    s = jnp.sum(p_ref[...], axis=1)
    m = jnp.max(s)
    e = jnp.exp(s - m)
    o_ref[...] = e / jnp.sum(e)


@jax.jit
def kernel(probs):
    partials = _sc_segsum(probs)
    return pl.pallas_call(
        _finish_body,
        out_shape=jax.ShapeDtypeStruct((_NA,), jnp.float32),
    )(partials)


# speedup vs baseline: 1.4936x; 1.4936x over previous
import functools

import jax
import jax.numpy as jnp
from jax import lax
from jax.experimental import pallas as pl
from jax.experimental.pallas import tpu as pltpu
from jax.experimental.pallas import tpu_sc as plsc

_N = 1 << 24
_NA = 4
_BUCKET = _N // _NA

_TC_BLK = 1 << 21
_TC_GRID = 6
_TC_N = _TC_BLK * _TC_GRID
_BPB = _BUCKET // _TC_BLK

_NW = 32
_WCHUNK = (_N - _TC_N) // _NW
_TILE = 32768
_NT = _WCHUNK // _TILE
_UNROLL = 8

_mesh = plsc.VectorSubcoreMesh(core_axis_name="c", subcore_axis_name="s")


@functools.partial(
    pl.kernel,
    mesh=_mesh,
    out_type=jax.ShapeDtypeStruct((_NA, 128), jnp.float32),
    scratch_types=[
        pltpu.VMEM((2, _TILE), jnp.float32),
        pltpu.VMEM((16,), jnp.float32),
        pltpu.SemaphoreType.DMA,
        pltpu.SemaphoreType.DMA,
    ],
)
def _sc_segsum(probs_hbm, out_hbm, buf, accv, sem0, sem1):
    cid = lax.axis_index("c")
    sid = lax.axis_index("s")
    wid = cid * 16 + sid
    base = _TC_N + wid * _WCHUNK
    sems = (sem0, sem1)

    copies = [None, None]
    copies[0] = pltpu.async_copy(
        probs_hbm.at[pl.ds(base, _TILE)], buf.at[0], sems[0])

    acc = tuple(jnp.zeros((16,), jnp.float32) for _ in range(_UNROLL))

    def _tile_sum(d, acc):
        tile = buf.at[d]

        def body(it, a):
            o = it * (_UNROLL * 16)
            return tuple(
                a[j] + tile[pl.ds(o + j * 16, 16)] for j in range(_UNROLL))

        return lax.fori_loop(0, _TILE // (16 * _UNROLL), body, acc)

    for t in range(_NT):
        d = t % 2
        if t + 1 < _NT:
            nd = (t + 1) % 2
            copies[nd] = pltpu.async_copy(
                probs_hbm.at[pl.ds(base + (t + 1) * _TILE, _TILE)],
                buf.at[nd], sems[nd])
        copies[d].wait()
        acc = _tile_sum(d, acc)

    vec = acc[0]
    for j in range(1, _UNROLL):
        vec = vec + acc[j]
    accv[...] = vec

    pltpu.sync_copy(accv, out_hbm.at[wid // 8, pl.ds((wid % 8) * 16, 16)])


def _tc_segsum_body(x_ref, o_ref):
    i = pl.program_id(0)

    @pl.when(i == 0)
    def _init():
        o_ref[...] = jnp.zeros_like(o_ref)

    blk = x_ref[...].reshape(_TC_BLK // 128, 128)
    partial = jnp.sum(blk, axis=0, keepdims=True)
    b = i // _BPB
    row = lax.broadcasted_iota(jnp.int32, (_NA, 128), 0)
    o_ref[...] += jnp.where(row == b, partial, 0.0)


def _combine_body(tc_ref, sc_ref, o_ref):
    s = jnp.sum(tc_ref[...], axis=1)
    sc_total = jnp.sum(sc_ref[...])
    idx = lax.broadcasted_iota(jnp.int32, (_NA,), 0)
    s = s + jnp.where(idx == _NA - 1, sc_total, 0.0)
    m = jnp.max(s)
    e = jnp.exp(s - m)
    o_ref[...] = e / jnp.sum(e)


@jax.jit
def kernel(probs):
    sc_partials = _sc_segsum(probs)
    tc_partials = pl.pallas_call(
        _tc_segsum_body,
        grid=(_TC_GRID,),
        in_specs=[pl.BlockSpec((_TC_BLK,), lambda i: (i,))],
        out_specs=pl.BlockSpec((_NA, 128), lambda i: (0, 0)),
        out_shape=jax.ShapeDtypeStruct((_NA, 128), jnp.float32),
        compiler_params=pltpu.CompilerParams(
            dimension_semantics=("arbitrary",),
        ),
    )(probs)
    return pl.pallas_call(
        _combine_body,
        out_shape=jax.ShapeDtypeStruct((_NA,), jnp.float32),
    )(tc_partials, sc_partials)


# speedup vs baseline: 1.4952x; 1.0011x over previous
import functools

import jax
import jax.numpy as jnp
from jax import lax
from jax.experimental import pallas as pl
from jax.experimental.pallas import tpu as pltpu
from jax.experimental.pallas import tpu_sc as plsc

_N = 1 << 24
_NA = 4
_BUCKET = _N // _NA

_TC_BLK = 1 << 21
_TC_GRID = 6
_TC_N = _TC_BLK * _TC_GRID
_BPB = _BUCKET // _TC_BLK

_NW = 32
_WCHUNK = (_N - _TC_N) // _NW
_TILE = 32768
_NT = _WCHUNK // _TILE
_UNROLL = 8

_mesh = plsc.VectorSubcoreMesh(core_axis_name="c", subcore_axis_name="s")


@functools.partial(
    pl.kernel,
    mesh=_mesh,
    out_type=jax.ShapeDtypeStruct((_NA, 128), jnp.float32),
    scratch_types=[
        pltpu.VMEM((2, _TILE), jnp.float32),
        pltpu.VMEM((16,), jnp.float32),
        pltpu.SemaphoreType.DMA,
        pltpu.SemaphoreType.DMA,
    ],
)
def _sc_segsum(probs_hbm, out_hbm, buf, accv, sem0, sem1):
    cid = lax.axis_index("c")
    sid = lax.axis_index("s")
    wid = cid * 16 + sid
    base = _TC_N + wid * _WCHUNK
    sems = (sem0, sem1)

    copies = [None, None]
    copies[0] = pltpu.async_copy(
        probs_hbm.at[pl.ds(base, _TILE)], buf.at[0], sems[0])

    acc = tuple(jnp.zeros((16,), jnp.float32) for _ in range(_UNROLL))

    def _tile_sum(d, acc):
        tile = buf.at[d]

        def body(it, a):
            o = it * (_UNROLL * 16)
            return tuple(
                a[j] + tile[pl.ds(o + j * 16, 16)] for j in range(_UNROLL))

        return lax.fori_loop(0, _TILE // (16 * _UNROLL), body, acc)

    for t in range(_NT):
        d = t % 2
        if t + 1 < _NT:
            nd = (t + 1) % 2
            copies[nd] = pltpu.async_copy(
                probs_hbm.at[pl.ds(base + (t + 1) * _TILE, _TILE)],
                buf.at[nd], sems[nd])
        copies[d].wait()
        acc = _tile_sum(d, acc)

    vec = acc[0]
    for j in range(1, _UNROLL):
        vec = vec + acc[j]
    accv[...] = vec

    pltpu.sync_copy(accv, out_hbm.at[wid // 8, pl.ds((wid % 8) * 16, 16)])


def _tc_segsum_body(x_ref, o_ref):
    i = pl.program_id(0)

    @pl.when(i == 0)
    def _init():
        o_ref[...] = jnp.zeros_like(o_ref)

    blk = x_ref[...].reshape(_TC_BLK // 128, 128)
    partial = jnp.sum(blk, axis=0, keepdims=True)
    b = i // _BPB
    row = lax.broadcasted_iota(jnp.int32, (_NA, 128), 0)
    o_ref[...] += jnp.where(row == b, partial, 0.0)


def _combine_body(tc_ref, sc_ref, o_ref):
    s = jnp.sum(tc_ref[...], axis=1)
    sc_total = jnp.sum(sc_ref[...])
    idx = lax.broadcasted_iota(jnp.int32, (_NA,), 0)
    s = s + jnp.where(idx == _NA - 1, sc_total, 0.0)
    m = jnp.max(s)
    e = jnp.exp(s - m)
    o_ref[...] = e / jnp.sum(e)


@jax.jit
def kernel(probs):
    tc_partials = pl.pallas_call(
        _tc_segsum_body,
        grid=(_TC_GRID,),
        in_specs=[pl.BlockSpec((_TC_BLK,), lambda i: (i,))],
        out_specs=pl.BlockSpec((_NA, 128), lambda i: (0, 0)),
        out_shape=jax.ShapeDtypeStruct((_NA, 128), jnp.float32),
        compiler_params=pltpu.CompilerParams(
            dimension_semantics=("arbitrary",),
        ),
    )(probs)
    sc_partials = _sc_segsum(probs)
    return pl.pallas_call(
        _combine_body,
        out_shape=jax.ShapeDtypeStruct((_NA,), jnp.float32),
    )(tc_partials, sc_partials)
